# baseline (speedup 1.0000x reference)
import functools

import jax
import jax.numpy as jnp
from jax import lax
from jax.experimental import pallas as pl
from jax.experimental.pallas import tpu as pltpu
from jax.experimental.pallas import tpu_sc as plsc

N = 10000
E = 320000
IN_CH = 128
HID = 256
NUM_GRAPHS = 64

NTILES = 16
CHUNK = 80
TCH = 256
EPAD = NTILES * TCH * CHUNK
SUPB = 32
NBLK = TCH // SUPB
NBLK0 = NBLK // 2
NBUF = 4
GRP = SUPB // NBUF
NPAD = N + 16
NTA = 624
NTAIL = N - NTILES * NTA

_F32 = jnp.float32
_HIGH = lax.Precision.HIGHEST


def _make_sc_agg():
  mesh = plsc.VectorSubcoreMesh(core_axis_name="c", subcore_axis_name="s")
  half = 128

  @functools.partial(
      pl.kernel,
      mesh=mesh,
      out_type=jax.ShapeDtypeStruct((2 * N, half), _F32),
      scratch_types=[
          pltpu.VMEM((SUPB, CHUNK), jnp.int32),
          pltpu.VMEM((SUPB, CHUNK), jnp.int32),
          pltpu.VMEM((NBUF, CHUNK, half), _F32),
          pltpu.VMEM((16,), jnp.int32),
          pltpu.VMEM_SHARED((NPAD, half), _F32),
          pltpu.SemaphoreType.DMA,
          pltpu.SemaphoreType.DMA,
      ],
  )
  def k(table, initsrc, srcpl, dstpl, cfg, z,
        src_v, dst_v, gbuf, cfg_v, agg, gsem, ssem):
    c = lax.axis_index("c")
    s = lax.axis_index("s")
    pltpu.sync_copy(cfg, cfg_v)
    nblk = cfg_v[...][0]
    pltpu.sync_copy(initsrc.at[pl.ds(c * N + s * NTA, NTA)],
                    agg.at[pl.ds(s * NTA, NTA)])

    @pl.when(s == NTILES - 1)
    def _init_tail():
      pltpu.sync_copy(initsrc.at[pl.ds(c * N + NTILES * NTA, NTAIL)],
                      agg.at[pl.ds(NTILES * NTA, NTAIL)])

    plsc.subcore_barrier()

    def start_gather(j, i):
      pltpu.async_copy(table.at[src_v.at[j]], gbuf.at[i], gsem)

    def wait_gather(i):
      pltpu.make_async_copy(table.at[src_v.at[0]], gbuf.at[i], gsem).wait()

    def start_scatter(j, i):
      pltpu.async_copy(gbuf.at[i], agg.at[dst_v.at[j]], ssem, add=True)

    def wait_scatter(i):
      pltpu.make_async_copy(gbuf.at[i], agg.at[dst_v.at[0]], ssem).wait()

    def outer(b, carry):
      _run_block(b)
      return carry

    def _run_block(b):
      pltpu.sync_copy(srcpl.at[c, s, pl.ds(b * SUPB, SUPB)], src_v)
      pltpu.sync_copy(dstpl.at[c, s, pl.ds(b * SUPB, SUPB)], dst_v)
      for t in range(NBUF - 2):
        start_gather(t, t)

      def group(g, carry2):
        for i in range(NBUF):
          j = g * NBUF + i
          jn = j + NBUF - 2
          bn = (i - 2) % NBUF
          wait_gather(i)
          start_scatter(j, i)
          if i < 2:
            @pl.when(g > 0)
            def _():
              wait_scatter(bn)

            start_gather(jn, bn)
          else:
            wait_scatter(bn)

            @pl.when(g < GRP - 1)
            def _():
              start_gather(jn, bn)
        return carry2

      lax.fori_loop(0, GRP, group, 0)
      wait_scatter((SUPB - 2) % NBUF)
      wait_scatter((SUPB - 1) % NBUF)

    lax.fori_loop(0, nblk, outer, 0)
    plsc.subcore_barrier()
    pltpu.sync_copy(agg.at[pl.ds(s * NTA, NTA)],
                    z.at[pl.ds(c * N + s * NTA, NTA)])

    @pl.when(s == NTILES - 1)
    def _wb_tail():
      pltpu.sync_copy(agg.at[pl.ds(NTILES * NTA, NTAIL)],
                      z.at[pl.ds(c * N + NTILES * NTA, NTAIL)])

  return k


def _mlp(z_st, W1, b1, W2, b2):
  half = HID // 2
  R = 1000
  nb = N // R

  def body(z_ref, w1_ref, b1_ref, w2_ref, b2_ref, o_ref):
    z = jnp.concatenate([z_ref[0], z_ref[1]], axis=1)
    h = jnp.dot(z, w1_ref[...], precision=_HIGH, preferred_element_type=_F32)
    h = jnp.maximum(h + b1_ref[...], 0.0)
    o = jnp.dot(h, w2_ref[...], precision=_HIGH, preferred_element_type=_F32)
    o = jnp.maximum(o + b2_ref[...], 0.0)
    o_ref[0] = o[:, :half]
    o_ref[1] = o[:, half:]

  return pl.pallas_call(
      body,
      grid=(nb,),
      in_specs=[
          pl.BlockSpec((2, R, half), lambda i: (0, i, 0)),
          pl.BlockSpec((HID, HID), lambda i: (0, 0)),
          pl.BlockSpec((1, HID), lambda i: (0, 0)),
          pl.BlockSpec((HID, HID), lambda i: (0, 0)),
          pl.BlockSpec((1, HID), lambda i: (0, 0)),
      ],
      out_specs=pl.BlockSpec((2, R, half), lambda i: (0, i, 0)),
      out_shape=jax.ShapeDtypeStruct((2, N, half), _F32),
  )(z_st, W1, b1, W2, b2)


def _pool_encode(h_st, batch_row, enc_Wout, enc_bout):
  R = 1000
  nb = N // R

  def body(h_ref, b_ref, w_ref, bias_ref, gemb_ref, pooled_acc, counts_acc):
    i = pl.program_id(0)

    @pl.when(i == 0)
    def _init():
      pooled_acc[...] = jnp.zeros_like(pooled_acc)
      counts_acc[...] = jnp.zeros_like(counts_acc)

    h = jnp.concatenate([h_ref[0], h_ref[1]], axis=1)
    gids = lax.broadcasted_iota(jnp.int32, (NUM_GRAPHS, R), 0)
    bmat_t = (b_ref[0] == gids).astype(_F32)
    pooled_acc[...] += jnp.dot(bmat_t, h, precision=_HIGH,
                               preferred_element_type=_F32)
    counts_acc[...] += jnp.sum(bmat_t, axis=1, keepdims=True)

    @pl.when(i == nb - 1)
    def _fin():
      pooled = pooled_acc[...] / jnp.maximum(counts_acc[...], 1.0)
      gemb_ref[...] = jnp.dot(pooled, w_ref[...], precision=_HIGH,
                              preferred_element_type=_F32) + bias_ref[...]

  return pl.pallas_call(
      body,
      grid=(nb,),
      in_specs=[
          pl.BlockSpec((2, R, HID // 2), lambda i: (0, i, 0)),
          pl.BlockSpec((1, 1, R), lambda i: (i, 0, 0)),
          pl.BlockSpec((HID, HID), lambda i: (0, 0)),
          pl.BlockSpec((1, HID), lambda i: (0, 0)),
      ],
      out_specs=pl.BlockSpec((NUM_GRAPHS, HID), lambda i: (0, 0)),
      out_shape=jax.ShapeDtypeStruct((NUM_GRAPHS, HID), _F32),
      scratch_shapes=[
          pltpu.VMEM((NUM_GRAPHS, HID), _F32),
          pltpu.VMEM((NUM_GRAPHS, 1), _F32),
      ],
  )(h_st, batch_row, enc_Wout, enc_bout.reshape(1, -1))


def _decode(gemb, batch_col, dec_W1, dec_b1, dec_W2, dec_b2):
  R = 1000
  nb = N // R

  def body(g_ref, b_ref, w1_ref, b1_ref, w2_ref, b2_ref, rec_ref):
    gids = lax.broadcasted_iota(jnp.int32, (R, NUM_GRAPHS), 1)
    bmat = (b_ref[...] == gids).astype(_F32)
    ne = jnp.dot(bmat, g_ref[...], precision=_HIGH,
                 preferred_element_type=_F32)
    hid = jnp.dot(ne, w1_ref[...], precision=_HIGH,
                  preferred_element_type=_F32)
    hid = jnp.maximum(hid + b1_ref[...], 0.0)
    rec = jnp.dot(hid, w2_ref[...], precision=_HIGH,
                  preferred_element_type=_F32)
    rec_ref[...] = rec + b2_ref[...]

  return pl.pallas_call(
      body,
      grid=(nb,),
      in_specs=[
          pl.BlockSpec((NUM_GRAPHS, HID), lambda i: (0, 0)),
          pl.BlockSpec((R, 1), lambda i: (i, 0)),
          pl.BlockSpec((HID, HID // 2), lambda i: (0, 0)),
          pl.BlockSpec((1, HID // 2), lambda i: (0, 0)),
          pl.BlockSpec((HID // 2, IN_CH), lambda i: (0, 0)),
          pl.BlockSpec((1, IN_CH), lambda i: (0, 0)),
      ],
      out_specs=pl.BlockSpec((R, IN_CH), lambda i: (i, 0)),
      out_shape=jax.ShapeDtypeStruct((N, IN_CH), _F32),
  )(gemb, batch_col, dec_W1, dec_b1.reshape(1, -1), dec_W2,
    dec_b2.reshape(1, -1))


_sc_agg_chan_split = _make_sc_agg()


def kernel(x, edge_index, batch,
           gin0_W1, gin0_b1, gin0_W2, gin0_b2,
           gin1_W1, gin1_b1, gin1_W2, gin1_b2,
           gin2_W1, gin2_b1, gin2_W2, gin2_b2,
           enc_Wout, enc_bout, dec_W1, dec_b1, dec_W2, dec_b2):
  src = jnp.concatenate(
      [edge_index[0], jnp.zeros((EPAD - E,), jnp.int32)])
  dst = jnp.concatenate(
      [edge_index[1], jnp.full((EPAD - E,), N, jnp.int32)])
  srcpl12 = jnp.stack([src, src + N]).reshape(2, NTILES, TCH, CHUNK)
  dstpl12 = jnp.stack([dst, dst]).reshape(2, NTILES, TCH, CHUNK)

  pt = E // 2 // NTILES
  ptp = NBLK0 * SUPB * CHUNK

  def _l0(idx, padval):
    a = idx.reshape(2, NTILES, pt)
    a = jnp.pad(a, ((0, 0), (0, 0), (0, ptp - pt)), constant_values=padval)
    a = a.reshape(2, NTILES, NBLK0 * SUPB, CHUNK)
    return jnp.pad(a, ((0, 0), (0, 0), (0, TCH - NBLK0 * SUPB), (0, 0)))

  srcpl0 = _l0(edge_index[0], 0)
  dstpl0 = _l0(edge_index[1], N)
  srcpls = jnp.stack([srcpl0, srcpl12, srcpl12])
  dstpls = jnp.stack([dstpl0, dstpl12, dstpl12])
  cfgs = jnp.full((3, 16), NBLK, jnp.int32).at[0].set(NBLK0)

  init0 = jnp.concatenate([x, jnp.zeros_like(x)], axis=0)
  h_st = jnp.stack([x, x])
  W1s = jnp.stack([
      jnp.concatenate([gin0_W1, gin0_W1], axis=0), gin1_W1, gin2_W1])
  b1s = jnp.stack([gin0_b1, gin1_b1, gin2_b1])[:, None, :]
  W2s = jnp.stack([gin0_W2, gin1_W2, gin2_W2])
  b2s = jnp.stack([gin0_b2, gin1_b2, gin2_b2])[:, None, :]

  def layer(h_st, ws):
    W1, b1, W2, b2, spl, dpl, cfg = ws
    table = h_st.reshape(2 * N, HID // 2)
    initsrc = jnp.where(cfg[0] == NBLK0, init0, table)
    z = _sc_agg_chan_split(table, initsrc, spl, dpl, cfg)
    return _mlp(z.reshape(2, N, HID // 2), W1, b1, W2, b2), None

  h_st, _ = lax.scan(layer, h_st, (W1s, b1s, W2s, b2s, srcpls, dstpls, cfgs))

  gemb = _pool_encode(h_st, batch.reshape(N // 1000, 1, 1000), enc_Wout,
                      enc_bout)
  rec = _decode(gemb, batch.reshape(N, 1), dec_W1, dec_b1, dec_W2, dec_b2)
  return (rec, gemb)


# speedup vs baseline: 1.0505x; 1.0505x over previous
import functools

import jax
import jax.numpy as jnp
from jax import lax
from jax.experimental import pallas as pl
from jax.experimental.pallas import tpu as pltpu
from jax.experimental.pallas import tpu_sc as plsc

N = 10000
E = 320000
IN_CH = 128
HID = 256
NUM_GRAPHS = 64

NTILES = 16
CHUNK = 64
TCH = 320
EPAD = NTILES * TCH * CHUNK
SUPB = 32
NBLK = TCH // SUPB
NBLK0 = 5
NBUF = 4
GRP = SUPB // NBUF
NPAD = N + 16
NTA = 624
NTAIL = N - NTILES * NTA

_F32 = jnp.float32
_HIGH = lax.Precision.HIGHEST


def _make_sc_agg():
  mesh = plsc.VectorSubcoreMesh(core_axis_name="c", subcore_axis_name="s")
  half = 128

  @functools.partial(
      pl.kernel,
      mesh=mesh,
      out_type=jax.ShapeDtypeStruct((2 * N, half), _F32),
      scratch_types=[
          pltpu.VMEM((2, SUPB, CHUNK), jnp.int32),
          pltpu.VMEM((2, SUPB, CHUNK), jnp.int32),
          pltpu.VMEM((NBUF, CHUNK, half), _F32),
          pltpu.VMEM((16,), jnp.int32),
          pltpu.VMEM_SHARED((NPAD, half), _F32),
          pltpu.SemaphoreType.DMA,
          pltpu.SemaphoreType.DMA,
          pltpu.SemaphoreType.DMA,
          pltpu.SemaphoreType.DMA,
      ],
  )
  def k(table, initsrc, srcpl, dstpl, cfg, z,
        src_v, dst_v, gbuf, cfg_v, agg, gsem, ssem, isem0, isem1):
    c = lax.axis_index("c")
    s = lax.axis_index("s")
    pltpu.sync_copy(cfg, cfg_v)
    nblk = cfg_v[...][0]
    pltpu.sync_copy(initsrc.at[pl.ds(c * N + s * NTA, NTA)],
                    agg.at[pl.ds(s * NTA, NTA)])

    @pl.when(s == NTILES - 1)
    def _init_tail():
      pltpu.sync_copy(initsrc.at[pl.ds(c * N + NTILES * NTA, NTAIL)],
                      agg.at[pl.ds(NTILES * NTA, NTAIL)])

    plsc.subcore_barrier()

    def load_idx(slot, b, sem):
      pltpu.async_copy(srcpl.at[c, s, pl.ds(b * SUPB, SUPB)],
                       src_v.at[slot], sem)
      pltpu.async_copy(dstpl.at[c, s, pl.ds(b * SUPB, SUPB)],
                       dst_v.at[slot], sem)

    def wait_idx(slot, sem):
      pltpu.make_async_copy(srcpl.at[c, s, pl.ds(0, SUPB)],
                            src_v.at[slot], sem).wait()
      pltpu.make_async_copy(dstpl.at[c, s, pl.ds(0, SUPB)],
                            dst_v.at[slot], sem).wait()

    def process(slot):
      sv = src_v.at[slot]
      dv = dst_v.at[slot]

      def start_gather(j, i):
        pltpu.async_copy(table.at[sv.at[j]], gbuf.at[i], gsem)

      def wait_gather(i):
        pltpu.make_async_copy(table.at[sv.at[0]], gbuf.at[i], gsem).wait()

      def start_scatter(j, i):
        pltpu.async_copy(gbuf.at[i], agg.at[dv.at[j]], ssem, add=True)

      def wait_scatter(i):
        pltpu.make_async_copy(gbuf.at[i], agg.at[dv.at[0]], ssem).wait()

      for t in range(NBUF - 2):
        start_gather(t, t)

      def group(g, carry2):
        for i in range(NBUF):
          j = g * NBUF + i
          jn = j + NBUF - 2
          bn = (i - 2) % NBUF
          wait_gather(i)
          start_scatter(j, i)
          if i < 2:
            @pl.when(g > 0)
            def _():
              wait_scatter(bn)

            start_gather(jn, bn)
          else:
            wait_scatter(bn)

            @pl.when(g < GRP - 1)
            def _():
              start_gather(jn, bn)
        return carry2

      lax.fori_loop(0, GRP, group, 0)
      wait_scatter((SUPB - 2) % NBUF)
      wait_scatter((SUPB - 1) % NBUF)

    load_idx(0, 0, isem0)
    wait_idx(0, isem0)

    @pl.when(1 < nblk)
    def _prime():
      load_idx(1, 1, isem1)

    def pair(p, carry):
      b0 = 2 * p
      b1 = b0 + 1

      @pl.when(p > 0)
      def _():
        wait_idx(0, isem0)

      process(0)

      @pl.when(b0 + 2 < nblk)
      def _():
        load_idx(0, b0 + 2, isem0)

      @pl.when(b1 < nblk)
      def _():
        wait_idx(1, isem1)
        process(1)

      @pl.when(b1 + 2 < nblk)
      def _():
        load_idx(1, b1 + 2, isem1)

      return carry

    lax.fori_loop(0, (nblk + 1) // 2, pair, 0)
    plsc.subcore_barrier()
    pltpu.sync_copy(agg.at[pl.ds(s * NTA, NTA)],
                    z.at[pl.ds(c * N + s * NTA, NTA)])

    @pl.when(s == NTILES - 1)
    def _wb_tail():
      pltpu.sync_copy(agg.at[pl.ds(NTILES * NTA, NTAIL)],
                      z.at[pl.ds(c * N + NTILES * NTA, NTAIL)])

  return k


def _mlp(z_st, W1, b1, W2, b2):
  half = HID // 2
  R = 1000
  nb = N // R

  def body(z_ref, w1_ref, b1_ref, w2_ref, b2_ref, o_ref):
    z = jnp.concatenate([z_ref[0], z_ref[1]], axis=1)
    h = jnp.dot(z, w1_ref[...], precision=_HIGH, preferred_element_type=_F32)
    h = jnp.maximum(h + b1_ref[...], 0.0)
    o = jnp.dot(h, w2_ref[...], precision=_HIGH, preferred_element_type=_F32)
    o = jnp.maximum(o + b2_ref[...], 0.0)
    o_ref[0] = o[:, :half]
    o_ref[1] = o[:, half:]

  return pl.pallas_call(
      body,
      grid=(nb,),
      in_specs=[
          pl.BlockSpec((2, R, half), lambda i: (0, i, 0)),
          pl.BlockSpec((HID, HID), lambda i: (0, 0)),
          pl.BlockSpec((1, HID), lambda i: (0, 0)),
          pl.BlockSpec((HID, HID), lambda i: (0, 0)),
          pl.BlockSpec((1, HID), lambda i: (0, 0)),
      ],
      out_specs=pl.BlockSpec((2, R, half), lambda i: (0, i, 0)),
      out_shape=jax.ShapeDtypeStruct((2, N, half), _F32),
  )(z_st, W1, b1, W2, b2)


def _pool_encode(h_st, batch_row, enc_Wout, enc_bout):
  R = 1000
  nb = N // R

  def body(h_ref, b_ref, w_ref, bias_ref, gemb_ref, pooled_acc, counts_acc):
    i = pl.program_id(0)

    @pl.when(i == 0)
    def _init():
      pooled_acc[...] = jnp.zeros_like(pooled_acc)
      counts_acc[...] = jnp.zeros_like(counts_acc)

    h = jnp.concatenate([h_ref[0], h_ref[1]], axis=1)
    gids = lax.broadcasted_iota(jnp.int32, (NUM_GRAPHS, R), 0)
    bmat_t = (b_ref[0] == gids).astype(_F32)
    pooled_acc[...] += jnp.dot(bmat_t, h, precision=_HIGH,
                               preferred_element_type=_F32)
    counts_acc[...] += jnp.sum(bmat_t, axis=1, keepdims=True)

    @pl.when(i == nb - 1)
    def _fin():
      pooled = pooled_acc[...] / jnp.maximum(counts_acc[...], 1.0)
      gemb_ref[...] = jnp.dot(pooled, w_ref[...], precision=_HIGH,
                              preferred_element_type=_F32) + bias_ref[...]

  return pl.pallas_call(
      body,
      grid=(nb,),
      in_specs=[
          pl.BlockSpec((2, R, HID // 2), lambda i: (0, i, 0)),
          pl.BlockSpec((1, 1, R), lambda i: (i, 0, 0)),
          pl.BlockSpec((HID, HID), lambda i: (0, 0)),
          pl.BlockSpec((1, HID), lambda i: (0, 0)),
      ],
      out_specs=pl.BlockSpec((NUM_GRAPHS, HID), lambda i: (0, 0)),
      out_shape=jax.ShapeDtypeStruct((NUM_GRAPHS, HID), _F32),
      scratch_shapes=[
          pltpu.VMEM((NUM_GRAPHS, HID), _F32),
          pltpu.VMEM((NUM_GRAPHS, 1), _F32),
      ],
  )(h_st, batch_row, enc_Wout, enc_bout.reshape(1, -1))


def _decode(gemb, batch_col, dec_W1, dec_b1, dec_W2, dec_b2):
  R = 1000
  nb = N // R

  def body(g_ref, b_ref, w1_ref, b1_ref, w2_ref, b2_ref, rec_ref):
    gids = lax.broadcasted_iota(jnp.int32, (R, NUM_GRAPHS), 1)
    bmat = (b_ref[...] == gids).astype(_F32)
    ne = jnp.dot(bmat, g_ref[...], precision=_HIGH,
                 preferred_element_type=_F32)
    hid = jnp.dot(ne, w1_ref[...], precision=_HIGH,
                  preferred_element_type=_F32)
    hid = jnp.maximum(hid + b1_ref[...], 0.0)
    rec = jnp.dot(hid, w2_ref[...], precision=_HIGH,
                  preferred_element_type=_F32)
    rec_ref[...] = rec + b2_ref[...]

  return pl.pallas_call(
      body,
      grid=(nb,),
      in_specs=[
          pl.BlockSpec((NUM_GRAPHS, HID), lambda i: (0, 0)),
          pl.BlockSpec((R, 1), lambda i: (i, 0)),
          pl.BlockSpec((HID, HID // 2), lambda i: (0, 0)),
          pl.BlockSpec((1, HID // 2), lambda i: (0, 0)),
          pl.BlockSpec((HID // 2, IN_CH), lambda i: (0, 0)),
          pl.BlockSpec((1, IN_CH), lambda i: (0, 0)),
      ],
      out_specs=pl.BlockSpec((R, IN_CH), lambda i: (i, 0)),
      out_shape=jax.ShapeDtypeStruct((N, IN_CH), _F32),
  )(gemb, batch_col, dec_W1, dec_b1.reshape(1, -1), dec_W2,
    dec_b2.reshape(1, -1))


_sc_agg_chan_split = _make_sc_agg()


def kernel(x, edge_index, batch,
           gin0_W1, gin0_b1, gin0_W2, gin0_b2,
           gin1_W1, gin1_b1, gin1_W2, gin1_b2,
           gin2_W1, gin2_b1, gin2_W2, gin2_b2,
           enc_Wout, enc_bout, dec_W1, dec_b1, dec_W2, dec_b2):
  src = jnp.concatenate(
      [edge_index[0], jnp.zeros((EPAD - E,), jnp.int32)])
  dst = jnp.concatenate(
      [edge_index[1], jnp.full((EPAD - E,), N, jnp.int32)])
  srcpl12 = jnp.stack([src, src + N]).reshape(2, NTILES, TCH, CHUNK)
  dstpl12 = jnp.stack([dst, dst]).reshape(2, NTILES, TCH, CHUNK)

  pt = E // 2 // NTILES
  ptp = NBLK0 * SUPB * CHUNK

  def _l0(idx, padval):
    a = idx.reshape(2, NTILES, pt)
    a = jnp.pad(a, ((0, 0), (0, 0), (0, ptp - pt)), constant_values=padval)
    a = a.reshape(2, NTILES, NBLK0 * SUPB, CHUNK)
    return jnp.pad(a, ((0, 0), (0, 0), (0, TCH - NBLK0 * SUPB), (0, 0)))

  srcpl0 = _l0(edge_index[0], 0)
  dstpl0 = _l0(edge_index[1], N)
  srcpls = jnp.stack([srcpl0, srcpl12, srcpl12])
  dstpls = jnp.stack([dstpl0, dstpl12, dstpl12])
  cfgs = jnp.full((3, 16), NBLK, jnp.int32).at[0].set(NBLK0)

  init0 = jnp.concatenate([x, jnp.zeros_like(x)], axis=0)
  h_st = jnp.stack([x, x])
  W1s = jnp.stack([
      jnp.concatenate([gin0_W1, gin0_W1], axis=0), gin1_W1, gin2_W1])
  b1s = jnp.stack([gin0_b1, gin1_b1, gin2_b1])[:, None, :]
  W2s = jnp.stack([gin0_W2, gin1_W2, gin2_W2])
  b2s = jnp.stack([gin0_b2, gin1_b2, gin2_b2])[:, None, :]

  def layer(h_st, ws):
    W1, b1, W2, b2, spl, dpl, cfg = ws
    table = h_st.reshape(2 * N, HID // 2)
    initsrc = jnp.where(cfg[0] == NBLK0, init0, table)
    z = _sc_agg_chan_split(table, initsrc, spl, dpl, cfg)
    return _mlp(z.reshape(2, N, HID // 2), W1, b1, W2, b2), None

  h_st, _ = lax.scan(layer, h_st, (W1s, b1s, W2s, b2s, srcpls, dstpls, cfgs))

  gemb = _pool_encode(h_st, batch.reshape(N // 1000, 1, 1000), enc_Wout,
                      enc_bout)
  rec = _decode(gemb, batch.reshape(N, 1), dec_W1, dec_b1, dec_W2, dec_b2)
  return (rec, gemb)


# speedup vs baseline: 1.0562x; 1.0054x over previous
import functools

import jax
import jax.numpy as jnp
from jax import lax
from jax.experimental import pallas as pl
from jax.experimental.pallas import tpu as pltpu
from jax.experimental.pallas import tpu_sc as plsc

N = 10000
E = 320000
IN_CH = 128
HID = 256
NUM_GRAPHS = 64

NTILES = 16
CHUNK = 64
TCH = 320
EPAD = NTILES * TCH * CHUNK
SUPB = 32
NBLK = TCH // SUPB
NBLK0 = 5
NBUF = 4
GRP = SUPB // NBUF
NPAD = N + 16
NTA = 624
NTAIL = N - NTILES * NTA

_F32 = jnp.float32
_HIGH = lax.Precision.HIGHEST


def _make_sc_agg():
  mesh = plsc.VectorSubcoreMesh(core_axis_name="c", subcore_axis_name="s")
  half = 128

  @functools.partial(
      pl.kernel,
      mesh=mesh,
      out_type=jax.ShapeDtypeStruct((2 * N, half), _F32),
      scratch_types=[
          pltpu.VMEM((2, SUPB, CHUNK), jnp.int32),
          pltpu.VMEM((2, SUPB, CHUNK), jnp.int32),
          pltpu.VMEM((NBUF, CHUNK, half), _F32),
          pltpu.VMEM((16,), jnp.int32),
          pltpu.VMEM_SHARED((NPAD, half), _F32),
          pltpu.SemaphoreType.DMA,
          pltpu.SemaphoreType.DMA,
          pltpu.SemaphoreType.DMA,
          pltpu.SemaphoreType.DMA,
      ],
  )
  def k(table, zeros, srcpl, dstpl, cfg, z,
        src_v, dst_v, gbuf, cfg_v, agg, gsem, ssem, isem0, isem1):
    c = lax.axis_index("c")
    s = lax.axis_index("s")
    pltpu.sync_copy(cfg, cfg_v)
    nblk = cfg_v[...][0]
    zinit = jnp.logical_and(nblk == NBLK0, c == 1)

    @pl.when(zinit)
    def _init_zero():
      pltpu.sync_copy(zeros.at[pl.ds(s * NTA, NTA)],
                      agg.at[pl.ds(s * NTA, NTA)])

      @pl.when(s == NTILES - 1)
      def _():
        pltpu.sync_copy(zeros.at[pl.ds(NTILES * NTA, NTAIL)],
                        agg.at[pl.ds(NTILES * NTA, NTAIL)])

    @pl.when(jnp.logical_not(zinit))
    def _init_h():
      pltpu.sync_copy(table.at[pl.ds(c * N + s * NTA, NTA)],
                      agg.at[pl.ds(s * NTA, NTA)])

      @pl.when(s == NTILES - 1)
      def _():
        pltpu.sync_copy(table.at[pl.ds(c * N + NTILES * NTA, NTAIL)],
                        agg.at[pl.ds(NTILES * NTA, NTAIL)])

    plsc.subcore_barrier()

    def load_idx(slot, b, sem):
      pltpu.async_copy(srcpl.at[c, s, pl.ds(b * SUPB, SUPB)],
                       src_v.at[slot], sem)
      pltpu.async_copy(dstpl.at[c, s, pl.ds(b * SUPB, SUPB)],
                       dst_v.at[slot], sem)

    def wait_idx(slot, sem):
      pltpu.make_async_copy(srcpl.at[c, s, pl.ds(0, SUPB)],
                            src_v.at[slot], sem).wait()
      pltpu.make_async_copy(dstpl.at[c, s, pl.ds(0, SUPB)],
                            dst_v.at[slot], sem).wait()

    def process(slot):
      sv = src_v.at[slot]
      dv = dst_v.at[slot]

      def start_gather(j, i):
        pltpu.async_copy(table.at[sv.at[j]], gbuf.at[i], gsem)

      def wait_gather(i):
        pltpu.make_async_copy(table.at[sv.at[0]], gbuf.at[i], gsem).wait()

      def start_scatter(j, i):
        pltpu.async_copy(gbuf.at[i], agg.at[dv.at[j]], ssem, add=True)

      def wait_scatter(i):
        pltpu.make_async_copy(gbuf.at[i], agg.at[dv.at[0]], ssem).wait()

      for t in range(NBUF - 2):
        start_gather(t, t)

      def group(g, carry2):
        for i in range(NBUF):
          j = g * NBUF + i
          jn = j + NBUF - 2
          bn = (i - 2) % NBUF
          wait_gather(i)
          start_scatter(j, i)
          if i < 2:
            @pl.when(g > 0)
            def _():
              wait_scatter(bn)

            start_gather(jn, bn)
          else:
            wait_scatter(bn)

            @pl.when(g < GRP - 1)
            def _():
              start_gather(jn, bn)
        return carry2

      lax.fori_loop(0, GRP, group, 0)
      wait_scatter((SUPB - 2) % NBUF)
      wait_scatter((SUPB - 1) % NBUF)

    load_idx(0, 0, isem0)
    wait_idx(0, isem0)

    @pl.when(1 < nblk)
    def _prime():
      load_idx(1, 1, isem1)

    def pair(p, carry):
      b0 = 2 * p
      b1 = b0 + 1

      @pl.when(p > 0)
      def _():
        wait_idx(0, isem0)

      process(0)

      @pl.when(b0 + 2 < nblk)
      def _():
        load_idx(0, b0 + 2, isem0)

      @pl.when(b1 < nblk)
      def _():
        wait_idx(1, isem1)
        process(1)

      @pl.when(b1 + 2 < nblk)
      def _():
        load_idx(1, b1 + 2, isem1)

      return carry

    lax.fori_loop(0, (nblk + 1) // 2, pair, 0)
    plsc.subcore_barrier()
    pltpu.sync_copy(agg.at[pl.ds(s * NTA, NTA)],
                    z.at[pl.ds(c * N + s * NTA, NTA)])

    @pl.when(s == NTILES - 1)
    def _wb_tail():
      pltpu.sync_copy(agg.at[pl.ds(NTILES * NTA, NTAIL)],
                      z.at[pl.ds(c * N + NTILES * NTA, NTAIL)])

  return k


def _mlp(z_st, W1, b1, W2, b2):
  half = HID // 2
  R = 1000
  nb = N // R

  def body(z_ref, w1_ref, b1_ref, w2_ref, b2_ref, o_ref):
    z = jnp.concatenate([z_ref[0], z_ref[1]], axis=1)
    h = jnp.dot(z, w1_ref[...], precision=_HIGH, preferred_element_type=_F32)
    h = jnp.maximum(h + b1_ref[...], 0.0)
    o = jnp.dot(h, w2_ref[...], precision=_HIGH, preferred_element_type=_F32)
    o = jnp.maximum(o + b2_ref[...], 0.0)
    o_ref[0] = o[:, :half]
    o_ref[1] = o[:, half:]

  return pl.pallas_call(
      body,
      grid=(nb,),
      in_specs=[
          pl.BlockSpec((2, R, half), lambda i: (0, i, 0)),
          pl.BlockSpec((HID, HID), lambda i: (0, 0)),
          pl.BlockSpec((1, HID), lambda i: (0, 0)),
          pl.BlockSpec((HID, HID), lambda i: (0, 0)),
          pl.BlockSpec((1, HID), lambda i: (0, 0)),
      ],
      out_specs=pl.BlockSpec((2, R, half), lambda i: (0, i, 0)),
      out_shape=jax.ShapeDtypeStruct((2, N, half), _F32),
  )(z_st, W1, b1, W2, b2)


def _pool_encode(h_st, batch_row, enc_Wout, enc_bout):
  R = 1000
  nb = N // R

  def body(h_ref, b_ref, w_ref, bias_ref, gemb_ref, pooled_acc, counts_acc):
    i = pl.program_id(0)

    @pl.when(i == 0)
    def _init():
      pooled_acc[...] = jnp.zeros_like(pooled_acc)
      counts_acc[...] = jnp.zeros_like(counts_acc)

    h = jnp.concatenate([h_ref[0], h_ref[1]], axis=1)
    gids = lax.broadcasted_iota(jnp.int32, (NUM_GRAPHS, R), 0)
    bmat_t = (b_ref[0] == gids).astype(_F32)
    pooled_acc[...] += jnp.dot(bmat_t, h, precision=_HIGH,
                               preferred_element_type=_F32)
    counts_acc[...] += jnp.sum(bmat_t, axis=1, keepdims=True)

    @pl.when(i == nb - 1)
    def _fin():
      pooled = pooled_acc[...] / jnp.maximum(counts_acc[...], 1.0)
      gemb_ref[...] = jnp.dot(pooled, w_ref[...], precision=_HIGH,
                              preferred_element_type=_F32) + bias_ref[...]

  return pl.pallas_call(
      body,
      grid=(nb,),
      in_specs=[
          pl.BlockSpec((2, R, HID // 2), lambda i: (0, i, 0)),
          pl.BlockSpec((1, 1, R), lambda i: (i, 0, 0)),
          pl.BlockSpec((HID, HID), lambda i: (0, 0)),
          pl.BlockSpec((1, HID), lambda i: (0, 0)),
      ],
      out_specs=pl.BlockSpec((NUM_GRAPHS, HID), lambda i: (0, 0)),
      out_shape=jax.ShapeDtypeStruct((NUM_GRAPHS, HID), _F32),
      scratch_shapes=[
          pltpu.VMEM((NUM_GRAPHS, HID), _F32),
          pltpu.VMEM((NUM_GRAPHS, 1), _F32),
      ],
  )(h_st, batch_row, enc_Wout, enc_bout.reshape(1, -1))


def _decode(gemb, batch_col, dec_W1, dec_b1, dec_W2, dec_b2):
  R = 1000
  nb = N // R

  def body(g_ref, b_ref, w1_ref, b1_ref, w2_ref, b2_ref, rec_ref):
    gids = lax.broadcasted_iota(jnp.int32, (R, NUM_GRAPHS), 1)
    bmat = (b_ref[...] == gids).astype(_F32)
    ne = jnp.dot(bmat, g_ref[...], precision=_HIGH,
                 preferred_element_type=_F32)
    hid = jnp.dot(ne, w1_ref[...], precision=_HIGH,
                  preferred_element_type=_F32)
    hid = jnp.maximum(hid + b1_ref[...], 0.0)
    rec = jnp.dot(hid, w2_ref[...], precision=_HIGH,
                  preferred_element_type=_F32)
    rec_ref[...] = rec + b2_ref[...]

  return pl.pallas_call(
      body,
      grid=(nb,),
      in_specs=[
          pl.BlockSpec((NUM_GRAPHS, HID), lambda i: (0, 0)),
          pl.BlockSpec((R, 1), lambda i: (i, 0)),
          pl.BlockSpec((HID, HID // 2), lambda i: (0, 0)),
          pl.BlockSpec((1, HID // 2), lambda i: (0, 0)),
          pl.BlockSpec((HID // 2, IN_CH), lambda i: (0, 0)),
          pl.BlockSpec((1, IN_CH), lambda i: (0, 0)),
      ],
      out_specs=pl.BlockSpec((R, IN_CH), lambda i: (i, 0)),
      out_shape=jax.ShapeDtypeStruct((N, IN_CH), _F32),
  )(gemb, batch_col, dec_W1, dec_b1.reshape(1, -1), dec_W2,
    dec_b2.reshape(1, -1))


_sc_agg_chan_split = _make_sc_agg()


def kernel(x, edge_index, batch,
           gin0_W1, gin0_b1, gin0_W2, gin0_b2,
           gin1_W1, gin1_b1, gin1_W2, gin1_b2,
           gin2_W1, gin2_b1, gin2_W2, gin2_b2,
           enc_Wout, enc_bout, dec_W1, dec_b1, dec_W2, dec_b2):
  src = jnp.concatenate(
      [edge_index[0], jnp.zeros((EPAD - E,), jnp.int32)])
  dst = jnp.concatenate(
      [edge_index[1], jnp.full((EPAD - E,), N, jnp.int32)])
  srcpl12 = jnp.stack([src, src + N]).reshape(2, NTILES, TCH, CHUNK)
  dstpl12 = jnp.stack([dst, dst]).reshape(2, NTILES, TCH, CHUNK)

  pt = E // 2 // NTILES
  ptp = NBLK0 * SUPB * CHUNK

  def _l0(idx, padval):
    a = idx.reshape(2, NTILES, pt)
    a = jnp.pad(a, ((0, 0), (0, 0), (0, ptp - pt)), constant_values=padval)
    a = a.reshape(2, NTILES, NBLK0 * SUPB, CHUNK)
    return jnp.pad(a, ((0, 0), (0, 0), (0, TCH - NBLK0 * SUPB), (0, 0)))

  srcpl0 = _l0(edge_index[0], 0)
  dstpl0 = _l0(edge_index[1], N)
  srcpls = jnp.stack([srcpl0, srcpl12, srcpl12])
  dstpls = jnp.stack([dstpl0, dstpl12, dstpl12])
  cfgs = jnp.full((3, 16), NBLK, jnp.int32).at[0].set(NBLK0)

  zeros = jnp.zeros((NPAD, IN_CH), _F32)
  h_st = jnp.stack([x, x])
  W1s = jnp.stack([
      jnp.concatenate([gin0_W1, gin0_W1], axis=0), gin1_W1, gin2_W1])
  b1s = jnp.stack([gin0_b1, gin1_b1, gin2_b1])[:, None, :]
  W2s = jnp.stack([gin0_W2, gin1_W2, gin2_W2])
  b2s = jnp.stack([gin0_b2, gin1_b2, gin2_b2])[:, None, :]

  def layer(h_st, ws):
    W1, b1, W2, b2, spl, dpl, cfg = ws
    table = h_st.reshape(2 * N, HID // 2)
    z = _sc_agg_chan_split(table, zeros, spl, dpl, cfg)
    return _mlp(z.reshape(2, N, HID // 2), W1, b1, W2, b2), None

  h_st, _ = lax.scan(layer, h_st, (W1s, b1s, W2s, b2s, srcpls, dstpls, cfgs))

  gemb = _pool_encode(h_st, batch.reshape(N // 1000, 1, 1000), enc_Wout,
                      enc_bout)
  rec = _decode(gemb, batch.reshape(N, 1), dec_W1, dec_b1, dec_W2, dec_b2)
  return (rec, gemb)
